# baseline (speedup 1.0000x reference)
import functools

import jax
import jax.numpy as jnp
from jax import lax
from jax.experimental import pallas as pl
from jax.experimental.pallas import tpu as pltpu
from jax.experimental.pallas import tpu_sc as plsc

N = 10000
E = 320000
D = 128

NPAD = 10240
EPAD = 327680
NC, NS = 2, 16
NW = NC * NS
EPT = EPAD // NW
CHUNK = 128
NCHUNK = EPT // CHUNK
RPT = NPAD // NS
CW = 8

def _mesh():
    return plsc.VectorSubcoreMesh(
        core_axis_name="c", subcore_axis_name="s", num_cores=NC, num_subcores=NS)


@functools.cache
def _sc_count_kernel():
    @functools.partial(
        pl.kernel,
        out_type=jax.ShapeDtypeStruct((NW, NPAD), jnp.float32),
        mesh=_mesh(),
        compiler_params=pltpu.CompilerParams(needs_layout_passes=False),
        scratch_types=[
            pltpu.VMEM((NCHUNK, CHUNK), jnp.int32),
            pltpu.VMEM((NPAD,), jnp.float32),
        ],
    )
    def _sc_count(dst_hbm, zeros_hbm, out_hbm, idx_v, cnt_v):
        c = lax.axis_index("c")
        s = lax.axis_index("s")
        w = c * NS + s
        pltpu.sync_copy(zeros_hbm, cnt_v)
        pltpu.sync_copy(dst_hbm.at[pl.ds(w * NCHUNK, NCHUNK)], idx_v)
        ones = jnp.full((16,), 1.0, jnp.float32)

        def body(j, carry):
            for k in range(CHUNK // 16):
                idx = idx_v[j, pl.ds(k * 16, 16)]
                plsc.addupdate_scatter(cnt_v, [idx], ones)
            return carry

        lax.fori_loop(0, NCHUNK, body, 0)
        pltpu.sync_copy(cnt_v, out_hbm.at[w])

    return _sc_count


def _sc_count(dst, zeros_n):
    return _sc_count_kernel()(dst, zeros_n)


NCH_ALL = EPAD // CHUNK
FAST_CORE = 0
QF = 112
QS = NCH_ALL // NS - QF
G = 8
PAIRS_F = QF // G // 2
PAIRS_S = QS // G // 2


@functools.cache
def _sc_agg_kernel():
    @functools.partial(
        pl.kernel,
        out_type=jax.ShapeDtypeStruct((NC, NPAD, D), jnp.float32),
        mesh=_mesh(),
        scratch_types=[
            pltpu.VMEM_SHARED((NPAD, D), jnp.float32),
            pltpu.SemaphoreType.DMA,
            pltpu.SemaphoreType.DMA,
        ],
    )
    def _sc_agg_k(y_hbm, src_hbm, dst_hbm, zeros_hbm, out_hbm, acc, sem, semi):
        pl.run_scoped(
            functools.partial(_agg_body, y_hbm, src_hbm, dst_hbm, zeros_hbm,
                              out_hbm, acc, sem, semi),
            pltpu.VMEM((G, CHUNK), jnp.int32),
            pltpu.VMEM((G, CHUNK), jnp.int32),
            pltpu.VMEM((G, CHUNK), jnp.int32),
            pltpu.VMEM((G, CHUNK), jnp.int32),
            pltpu.VMEM((CHUNK, D), jnp.float32),
            pltpu.VMEM((CHUNK, D), jnp.float32),
        )

    def _agg_body(y_hbm, src_hbm, dst_hbm, zeros_hbm, out_hbm, acc, sem, semi,
                  srcA, dstA, srcB, dstB, rows0, rows1):
        c = lax.axis_index("c")
        s = lax.axis_index("s")
        npairs = jnp.where(c == FAST_CORE, PAIRS_F, PAIRS_S)
        tb = jnp.where(c == FAST_CORE, s * QF, NS * QF + s * QS)
        pltpu.sync_copy(zeros_hbm.at[pl.ds(s * RPT, RPT)], acc.at[pl.ds(s * RPT, RPT)])
        pltpu.sync_copy(src_hbm.at[pl.ds(tb, G)], srcA)
        pltpu.sync_copy(dst_hbm.at[pl.ds(tb, G)], dstA)
        pltpu.async_copy(y_hbm.at[srcA.at[0]], rows0, sem)
        plsc.subcore_barrier()

        bufs = (rows0, rows1)

        def half(src_cur, dst_cur, src_nxt, dst_nxt, base_nxt, has_next):
            def when_next(fn):
                if has_next is True:
                    fn()
                else:
                    pl.when(has_next)(fn)

            for k in range(G):
                cur = bufs[k % 2]
                nxt = bufs[(k + 1) % 2]
                pltpu.make_async_copy(y_hbm.at[src_cur.at[k]], cur, sem).wait()
                if k == 0:
                    def fire_idx():
                        pltpu.async_copy(src_hbm.at[pl.ds(base_nxt, G)], src_nxt, semi)
                        pltpu.async_copy(dst_hbm.at[pl.ds(base_nxt, G)], dst_nxt, semi)
                    when_next(fire_idx)
                if k < G - 1:
                    pltpu.async_copy(y_hbm.at[src_cur.at[k + 1]], nxt, sem)
                else:
                    def wait_prime():
                        pltpu.make_async_copy(
                            src_hbm.at[pl.ds(base_nxt, G)], src_nxt, semi).wait()
                        pltpu.make_async_copy(
                            dst_hbm.at[pl.ds(base_nxt, G)], dst_nxt, semi).wait()
                        pltpu.async_copy(y_hbm.at[src_nxt.at[0]], nxt, sem)
                    when_next(wait_prime)
                pltpu.sync_copy(cur, acc.at[dst_cur.at[k]], add=True)

        def body(m, carry):
            baseA = tb + m * 2 * G
            baseB = baseA + G
            baseA2 = baseB + G
            not_last = m < npairs - 1
            half(srcA, dstA, srcB, dstB, baseB, True)
            half(srcB, dstB, srcA, dstA, baseA2, not_last)
            return carry

        lax.fori_loop(0, npairs, body, 0)
        plsc.subcore_barrier()
        pltpu.sync_copy(acc.at[pl.ds(s * RPT, RPT)], out_hbm.at[c, pl.ds(s * RPT, RPT)])

    return _sc_agg_k


def _sc_agg(y, src, dst, zeros_d):
    return _sc_agg_kernel()(y, src, dst, zeros_d)


BLK = 1024
GRID = NPAD // BLK


def _tc_first(feat_ref, w_ref, c_ref, y_ref, dis_ref):
    cnt = jnp.sum(c_ref[...], axis=0)
    dis = lax.rsqrt(cnt + 1.0)[:, None]
    xw = jnp.dot(feat_ref[...], w_ref[...], preferred_element_type=jnp.float32)
    y_ref[...] = xw * dis
    dis_ref[...] = jnp.broadcast_to(dis, (BLK, CW))


def _tc_mid(y_ref, p_ref, dis_ref, b_ref, w_ref, y2_ref):
    d = dis_ref[...][:, 0:1]
    z = jnp.maximum(d * (y_ref[...] + p_ref[0] + p_ref[1]) + b_ref[...], 0.0)
    y2_ref[...] = jnp.dot(z, w_ref[...], preferred_element_type=jnp.float32) * d


def _tc_last(y_ref, p_ref, dis_ref, b_ref, o_ref):
    d = dis_ref[...][:, 0:1]
    o_ref[...] = jnp.maximum(d * (y_ref[...] + p_ref[0] + p_ref[1]) + b_ref[...], 0.0)


def _row_spec(width):
    return pl.BlockSpec((BLK, width), lambda i: (i, 0))


def _pair_spec(width):
    return pl.BlockSpec((NC, BLK, width), lambda i: (0, i, 0))


def _full_spec(shape):
    return pl.BlockSpec(shape, lambda i: tuple(0 for _ in shape))


def kernel(feat, edge_index, W1, b1, W2, b2):
    src = edge_index[0].astype(jnp.int32)
    dst = edge_index[1].astype(jnp.int32)
    pad_e = EPAD - E
    src = jnp.concatenate([src, jnp.full((pad_e,), N, jnp.int32)]).reshape(NCH_ALL, CHUNK)
    dst = jnp.concatenate([dst, jnp.full((pad_e,), N, jnp.int32)]).reshape(NCH_ALL, CHUNK)
    featp = jnp.zeros((NPAD, D), jnp.float32).at[:N].set(feat)
    zeros_n = jnp.zeros((NPAD,), jnp.float32)
    zeros_d = jnp.zeros((NPAD, D), jnp.float32)
    b1r = b1.reshape(1, D)
    b2r = b2.reshape(1, D)

    cnt = _sc_count(dst, zeros_n)

    y1, dis = pl.pallas_call(
        _tc_first,
        grid=(GRID,),
        in_specs=[_row_spec(D), _full_spec((D, D)),
                  pl.BlockSpec((NW, BLK), lambda i: (0, i))],
        out_specs=[_row_spec(D), _row_spec(CW)],
        out_shape=[jax.ShapeDtypeStruct((NPAD, D), jnp.float32),
                   jax.ShapeDtypeStruct((NPAD, CW), jnp.float32)],
    )(featp, W1, cnt)

    p1 = _sc_agg(y1, src, dst, zeros_d)

    y2 = pl.pallas_call(
        _tc_mid,
        grid=(GRID,),
        in_specs=[_row_spec(D), _pair_spec(D), _row_spec(CW),
                  _full_spec((1, D)), _full_spec((D, D))],
        out_specs=_row_spec(D),
        out_shape=jax.ShapeDtypeStruct((NPAD, D), jnp.float32),
    )(y1, p1, dis, b1r, W2)

    p2 = _sc_agg(y2, src, dst, zeros_d)

    out = pl.pallas_call(
        _tc_last,
        grid=(GRID,),
        in_specs=[_row_spec(D), _pair_spec(D), _row_spec(CW), _full_spec((1, D))],
        out_specs=_row_spec(D),
        out_shape=jax.ShapeDtypeStruct((NPAD, D), jnp.float32),
    )(y2, p2, dis, b2r)

    return out[:N]


# speedup vs baseline: 1.0060x; 1.0060x over previous
import functools

import jax
import jax.numpy as jnp
from jax import lax
from jax.experimental import pallas as pl
from jax.experimental.pallas import tpu as pltpu
from jax.experimental.pallas import tpu_sc as plsc

N = 10000
E = 320000
D = 128

NPAD = 10240
EPAD = 327680
NC, NS = 2, 16
NW = NC * NS
EPT = EPAD // NW
CHUNK = 128
NCHUNK = EPT // CHUNK
RPT = NPAD // NS
CW = 8

def _mesh():
    return plsc.VectorSubcoreMesh(
        core_axis_name="c", subcore_axis_name="s", num_cores=NC, num_subcores=NS)


@functools.cache
def _sc_count_kernel():
    @functools.partial(
        pl.kernel,
        out_type=jax.ShapeDtypeStruct((NW, NPAD), jnp.float32),
        mesh=_mesh(),
        compiler_params=pltpu.CompilerParams(needs_layout_passes=False),
        scratch_types=[
            pltpu.VMEM((NCHUNK, CHUNK), jnp.int32),
            pltpu.VMEM((NPAD,), jnp.float32),
        ],
    )
    def _sc_count(dst_hbm, zeros_hbm, out_hbm, idx_v, cnt_v):
        c = lax.axis_index("c")
        s = lax.axis_index("s")
        w = c * NS + s
        pltpu.sync_copy(zeros_hbm, cnt_v)
        pltpu.sync_copy(dst_hbm.at[pl.ds(w * NCHUNK, NCHUNK)], idx_v)
        ones = jnp.full((16,), 1.0, jnp.float32)

        def body(j, carry):
            for k in range(CHUNK // 16):
                idx = idx_v[j, pl.ds(k * 16, 16)]
                plsc.addupdate_scatter(cnt_v, [idx], ones)
            return carry

        lax.fori_loop(0, NCHUNK, body, 0)
        pltpu.sync_copy(cnt_v, out_hbm.at[w])

    return _sc_count


def _sc_count(dst, zeros_n):
    return _sc_count_kernel()(dst, zeros_n)


NCH_ALL = EPAD // CHUNK
FAST_CORE = 0
QF = 112
QS = NCH_ALL // NS - QF
G = 8
PAIRS_F = QF // G // 2
PAIRS_S = QS // G // 2


@functools.cache
def _sc_agg_kernel():
    @functools.partial(
        pl.kernel,
        out_type=jax.ShapeDtypeStruct((NC, NPAD, D), jnp.float32),
        mesh=_mesh(),
        scratch_types=[
            pltpu.VMEM_SHARED((NPAD, D), jnp.float32),
            pltpu.SemaphoreType.DMA,
            pltpu.SemaphoreType.DMA,
        ],
    )
    def _sc_agg_k(y_hbm, src_hbm, dst_hbm, zeros_hbm, out_hbm, acc, sem, semi):
        pl.run_scoped(
            functools.partial(_agg_body, y_hbm, src_hbm, dst_hbm, zeros_hbm,
                              out_hbm, acc, sem, semi),
            pltpu.VMEM((G, CHUNK), jnp.int32),
            pltpu.VMEM((G, CHUNK), jnp.int32),
            pltpu.VMEM((G, CHUNK), jnp.int32),
            pltpu.VMEM((G, CHUNK), jnp.int32),
            pltpu.VMEM((CHUNK, D), jnp.float32),
            pltpu.VMEM((CHUNK, D), jnp.float32),
        )

    def _agg_body(y_hbm, src_hbm, dst_hbm, zeros_hbm, out_hbm, acc, sem, semi,
                  srcA, dstA, srcB, dstB, rows0, rows1):
        c = lax.axis_index("c")
        s = lax.axis_index("s")
        npairs = jnp.where(c == FAST_CORE, PAIRS_F, PAIRS_S)
        tb = jnp.where(c == FAST_CORE, s * QF, NS * QF + s * QS)
        pltpu.sync_copy(zeros_hbm.at[pl.ds(s * RPT, RPT)], acc.at[pl.ds(s * RPT, RPT)])
        pltpu.sync_copy(src_hbm.at[pl.ds(tb, G)], srcA)
        pltpu.sync_copy(dst_hbm.at[pl.ds(tb, G)], dstA)
        pltpu.async_copy(y_hbm.at[srcA.at[0]], rows0, sem)
        plsc.subcore_barrier()

        bufs = (rows0, rows1)

        def half(src_cur, dst_cur, src_nxt, dst_nxt, base_nxt, has_next):
            def when_next(fn):
                if has_next is True:
                    fn()
                else:
                    pl.when(has_next)(fn)

            for k in range(G):
                cur = bufs[k % 2]
                nxt = bufs[(k + 1) % 2]
                pltpu.make_async_copy(y_hbm.at[src_cur.at[k]], cur, sem).wait()
                if k == 0:
                    def fire_idx():
                        pltpu.async_copy(src_hbm.at[pl.ds(base_nxt, G)], src_nxt, semi)
                        pltpu.async_copy(dst_hbm.at[pl.ds(base_nxt, G)], dst_nxt, semi)
                    when_next(fire_idx)
                if k < G - 1:
                    pltpu.async_copy(y_hbm.at[src_cur.at[k + 1]], nxt, sem)
                else:
                    def wait_prime():
                        pltpu.make_async_copy(
                            src_hbm.at[pl.ds(base_nxt, G)], src_nxt, semi).wait()
                        pltpu.make_async_copy(
                            dst_hbm.at[pl.ds(base_nxt, G)], dst_nxt, semi).wait()
                        pltpu.async_copy(y_hbm.at[src_nxt.at[0]], nxt, sem)
                    when_next(wait_prime)
                pltpu.sync_copy(cur, acc.at[dst_cur.at[k]], add=True)

        def body(m, carry):
            baseA = tb + m * 2 * G
            baseB = baseA + G
            baseA2 = baseB + G
            not_last = m < npairs - 1
            half(srcA, dstA, srcB, dstB, baseB, True)
            half(srcB, dstB, srcA, dstA, baseA2, not_last)
            return carry

        lax.fori_loop(0, npairs, body, 0)
        plsc.subcore_barrier()
        pltpu.sync_copy(acc.at[pl.ds(s * RPT, RPT)], out_hbm.at[c, pl.ds(s * RPT, RPT)])

    return _sc_agg_k


def _sc_agg(y, src, dst, zeros_d):
    return _sc_agg_kernel()(y, src, dst, zeros_d)


BLK = 1024
GRID = NPAD // BLK


def _tc_first(feat_ref, w_ref, c_ref, y_ref, dis_ref):
    cnt = jnp.sum(c_ref[...], axis=0)
    dis = lax.rsqrt(cnt + 1.0)[:, None]
    xw = jnp.dot(feat_ref[...], w_ref[...], preferred_element_type=jnp.float32)
    y_ref[...] = xw * dis
    dis_ref[...] = jnp.broadcast_to(dis, (BLK, CW))


def _tc_mid(y_ref, p_ref, dis_ref, b_ref, w_ref, y2_ref):
    d = dis_ref[...][:, 0:1]
    z = jnp.maximum(d * (y_ref[...] + p_ref[0] + p_ref[1]) + b_ref[...], 0.0)
    y2_ref[...] = jnp.dot(z, w_ref[...], preferred_element_type=jnp.float32) * d


def _tc_last(y_ref, p_ref, dis_ref, b_ref, o_ref):
    d = dis_ref[...][:, 0:1]
    o_ref[...] = jnp.maximum(d * (y_ref[...] + p_ref[0] + p_ref[1]) + b_ref[...], 0.0)


def _row_spec(width):
    return pl.BlockSpec((BLK, width), lambda i: (i, 0))


def _pair_spec(width):
    return pl.BlockSpec((NC, BLK, width), lambda i: (0, i, 0))


def _full_spec(shape):
    return pl.BlockSpec(shape, lambda i: tuple(0 for _ in shape))


def kernel(feat, edge_index, W1, b1, W2, b2):
    src = edge_index[0].astype(jnp.int32)
    dst = edge_index[1].astype(jnp.int32)
    pad_e = EPAD - E
    pad_dst = N + (jnp.arange(pad_e, dtype=jnp.int32) % (NPAD - N))
    src = jnp.concatenate([src, jnp.full((pad_e,), N, jnp.int32)]).reshape(NCH_ALL, CHUNK)
    dst = jnp.concatenate([dst, pad_dst]).reshape(NCH_ALL, CHUNK)
    featp = jnp.zeros((NPAD, D), jnp.float32).at[:N].set(feat)
    zeros_n = jnp.zeros((NPAD,), jnp.float32)
    zeros_d = jnp.zeros((NPAD, D), jnp.float32)
    b1r = b1.reshape(1, D)
    b2r = b2.reshape(1, D)

    cnt = _sc_count(dst, zeros_n)

    y1, dis = pl.pallas_call(
        _tc_first,
        grid=(GRID,),
        in_specs=[_row_spec(D), _full_spec((D, D)),
                  pl.BlockSpec((NW, BLK), lambda i: (0, i))],
        out_specs=[_row_spec(D), _row_spec(CW)],
        out_shape=[jax.ShapeDtypeStruct((NPAD, D), jnp.float32),
                   jax.ShapeDtypeStruct((NPAD, CW), jnp.float32)],
    )(featp, W1, cnt)

    p1 = _sc_agg(y1, src, dst, zeros_d)

    y2 = pl.pallas_call(
        _tc_mid,
        grid=(GRID,),
        in_specs=[_row_spec(D), _pair_spec(D), _row_spec(CW),
                  _full_spec((1, D)), _full_spec((D, D))],
        out_specs=_row_spec(D),
        out_shape=jax.ShapeDtypeStruct((NPAD, D), jnp.float32),
    )(y1, p1, dis, b1r, W2)

    p2 = _sc_agg(y2, src, dst, zeros_d)

    out = pl.pallas_call(
        _tc_last,
        grid=(GRID,),
        in_specs=[_row_spec(D), _pair_spec(D), _row_spec(CW), _full_spec((1, D))],
        out_specs=_row_spec(D),
        out_shape=jax.ShapeDtypeStruct((NPAD, D), jnp.float32),
    )(y2, p2, dis, b2r)

    return out[:N]
